# baseline (speedup 1.0000x reference)
import functools

import jax
import jax.numpy as jnp
import numpy as np
from jax import lax
from jax.experimental import pallas as pl
from jax.experimental.pallas import tpu as pltpu
from jax.experimental.pallas import tpu_sc as plsc

B = 32
N_SC = 16
K_SPLIT = 32 // N_SC
TROWS = 576 // K_SPLIT
PP = 576
L = 16
R = 32
NCHUNK = TROWS // R
CPR = PP // L
GR = 8
DENOM = float(B * (PP * PP - PP))

C_OUT = np.float32(0.8)
ONE = np.float32(1.0)
FZERO = np.float32(0.0)
A0 = np.float32(0.8)
A1 = np.float32(-0.7)
B0 = np.float32(-0.7)
B1 = np.float32(1.5)

_BCAST = np.kron(np.eye(GR, dtype=np.float32), np.ones((1, L), np.float32))


def _body(real_hbm, fake_hbm, fw_hbm, g_hbm, out_hbm,
          rb0, rb1, fb0, fb1, gball, fwb, accb,
          sr0, sr1, sf0, sf1):
    info = plsc.get_sparse_core_info()
    nc = info.num_cores
    wid = lax.axis_index("s") * nc + lax.axis_index("c")
    bat = lax.div(wid, K_SPLIT)
    row0 = lax.rem(wid, K_SPLIT) * TROWS

    pltpu.sync_copy(fw_hbm.at[pl.ds(bat * PP, PP)], fwb)
    pltpu.sync_copy(g_hbm.at[pl.ds(bat * (PP // GR), PP // GR)], gball)
    g = []
    for c in range(CPR):
        w = fwb[pl.ds(c * L, L)]
        g.append(jnp.where(w > FZERO, ONE, FZERO))

    rbufs = (rb0, rb1)
    fbufs = (fb0, fb1)
    rsems = (sr0, sr1)
    fsems = (sf0, sf1)

    def start(ci):
        k = ci % 2
        r0 = row0 + ci * R
        cr = pltpu.async_copy(real_hbm.at[bat, pl.ds(r0, R)], rbufs[k], rsems[k])
        cf = pltpu.async_copy(fake_hbm.at[bat, pl.ds(r0, R)], fbufs[k], fsems[k])
        return cr, cf

    copies = [None] * NCHUNK
    copies[0] = start(0)

    zero = jnp.zeros((L,), jnp.float32)
    accs = (zero, zero, zero, zero)
    iota = lax.broadcasted_iota(jnp.int32, (L,), 0)

    for ci in range(NCHUNK):
        if ci + 1 < NCHUNK:
            copies[ci + 1] = start(ci + 1)
        for cp in copies[ci]:
            cp.wait()
        rb = rbufs[ci % 2]
        fb = fbufs[ci % 2]

        def rbody(r, a):
            aa = [a[0], a[1], a[2], a[3]]
            for c in range(CPR):
                aa[c % 4] = aa[c % 4] + jnp.abs(rb[r, pl.ds(c * L, L)] - C_OUT)
            return (aa[0], aa[1], aa[2], aa[3])

        accs = lax.fori_loop(0, R, rbody, accs)

        def fbody(r, a):
            a0, a1, a2, a3 = a
            i_glob = row0 + ci * R + r

            lane = lax.rem(i_glob, L)
            dv = rb[r, pl.ds(pl.multiple_of(i_glob - lane, L), L)]
            corr = jnp.abs(dv - ONE) - jnp.abs(dv - C_OUT)
            a0 = a0 + jnp.where(iota == lane, corr, FZERO)

            gi = gball[lax.div(i_glob, GR),
                       pl.ds(pl.multiple_of(lax.rem(i_glob, GR) * L, L), L)]
            av = A0 + A1 * gi
            bv = B0 + B1 * gi
            aa = [a0, a1, a2, a3]
            for c in range(CPR):
                x = fb[r, pl.ds(c * L, L)]
                t = av + bv * g[c]
                aa[c % 4] = aa[c % 4] + jnp.abs(x - t)
            return (aa[0], aa[1], aa[2], aa[3])

        accs = lax.fori_loop(0, R, fbody, accs)

    acc = (accs[0] + accs[1]) + (accs[2] + accs[3])
    accb[...] = acc
    pltpu.sync_copy(accb, out_hbm.at[wid])


def _sc_call(real, fake, fw_flat, g16):
    mesh = plsc.VectorSubcoreMesh(core_axis_name="c", subcore_axis_name="s")
    kfn = functools.partial(
        pl.kernel,
        mesh=mesh,
        out_type=jax.ShapeDtypeStruct((32, L), jnp.float32),
        scratch_types=[
            pltpu.VMEM((R, PP), jnp.float32),
            pltpu.VMEM((R, PP), jnp.float32),
            pltpu.VMEM((R, PP), jnp.float32),
            pltpu.VMEM((R, PP), jnp.float32),
            pltpu.VMEM((PP // GR, GR * L), jnp.float32),
            pltpu.VMEM((PP,), jnp.float32),
            pltpu.VMEM((L,), jnp.float32),
            pltpu.SemaphoreType.DMA,
            pltpu.SemaphoreType.DMA,
            pltpu.SemaphoreType.DMA,
            pltpu.SemaphoreType.DMA,
        ],
    )(_body)
    return kfn(real, fake, fw_flat, g16)


RB = 576
NRB = PP // RB
BB = 4


def _tc_body(fw_ref, real_ref, fake_ref, out_ref):
    b = pl.program_id(0)

    ii = lax.broadcasted_iota(jnp.int32, (RB, PP), 0)
    jj = lax.broadcasted_iota(jnp.int32, (RB, PP), 1)
    tr = jnp.where(ii == jj, ONE, C_OUT)
    bi = lax.broadcasted_iota(jnp.int32, (1, B), 1)
    fw = fw_ref[...]

    s = FZERO
    for u in range(BB):
        x = real_ref[u]
        s = s + jnp.sum(jnp.abs(x - tr))
        y = fake_ref[u]
        oh = jnp.where(bi == b * BB + u, ONE, FZERO)
        row = jax.lax.dot_general(
            oh, fw, (((1,), (0,)), ((), ())),
            preferred_element_type=jnp.float32)
        g1 = jnp.where(row > FZERO, ONE, FZERO)
        gj = g1.reshape(1, PP)
        gi = g1.reshape(RB, 1)
        t = A0 + A1 * (gi + gj) + np.float32(1.5) * (gi * gj)
        s = s + jnp.sum(jnp.abs(y - t))

    @pl.when(b == 0)
    def _init():
        out_ref[0, 0] = FZERO

    out_ref[0, 0] += s * np.float32(1.0 / DENOM)


def _tc_call(real, fake, fw3, gblk):
    n = B - N_SC
    return pl.pallas_call(
        _tc_body,
        grid=(n, NRB),
        in_specs=[
            pl.BlockSpec((1, 1, PP), lambda i, j: (i + N_SC, 0, 0)),
            pl.BlockSpec((1, 1, 1, RB), lambda i, j: (i + N_SC, j, 0, 0)),
            pl.BlockSpec((1, RB, PP), lambda i, j: (i + N_SC, j, 0)),
            pl.BlockSpec((1, RB, PP), lambda i, j: (i + N_SC, j, 0)),
        ],
        out_specs=pl.BlockSpec(
            (1, 1), lambda i, j: (0, 0), memory_space=pltpu.SMEM),
        out_shape=jax.ShapeDtypeStruct((1, 1), jnp.float32),
    )(fw3, gblk, real, fake)


@jax.jit
def _run(real, fake, fw):
    gvals = (fw > 0.0).astype(jnp.float32)
    fw3 = fw[:, None, :]
    gblk = gvals.reshape(B, NRB, 1, RB)
    fw_flat = fw.reshape(-1)
    g16 = gvals.reshape(B * PP // GR, GR) @ jnp.asarray(_BCAST)
    sc_parts = _sc_call(real, fake, fw_flat, g16)
    tc_sum = _tc_call(real, fake, fw3, gblk)
    return (jnp.sum(sc_parts) + tc_sum.reshape(())) / np.float32(DENOM)


@jax.jit
def _run_tc_only(real, fake, fw):
    tc_sum = _tc_call_all(real, fake, fw)
    return tc_sum.reshape(())


def _tc_call_all(real, fake, fw):
    return pl.pallas_call(
        _tc_body,
        grid=(B // BB,),
        in_specs=[
            pl.BlockSpec((B, PP), lambda i: (0, 0)),
            pl.BlockSpec((BB, RB, PP), lambda i: (i, 0, 0)),
            pl.BlockSpec((BB, RB, PP), lambda i: (i, 0, 0)),
        ],
        out_specs=pl.BlockSpec(
            (1, 1), lambda i: (0, 0), memory_space=pltpu.SMEM),
        out_shape=jax.ShapeDtypeStruct((1, 1), jnp.float32),
    )(fw, real, fake)


def kernel(correlation_map_real, correlation_map_fake, fake_weight):
    return _run_tc_only(correlation_map_real, correlation_map_fake, fake_weight)


# speedup vs baseline: 1.0629x; 1.0629x over previous
import functools

import jax
import jax.numpy as jnp
import numpy as np
from jax import lax
from jax.experimental import pallas as pl
from jax.experimental.pallas import tpu as pltpu
from jax.experimental.pallas import tpu_sc as plsc

B = 32
N_SC = 16
K_SPLIT = 32 // N_SC
TROWS = 576 // K_SPLIT
PP = 576
L = 16
R = 32
NCHUNK = TROWS // R
CPR = PP // L
GR = 8
DENOM = float(B * (PP * PP - PP))

C_OUT = np.float32(0.8)
ONE = np.float32(1.0)
FZERO = np.float32(0.0)
A0 = np.float32(0.8)
A1 = np.float32(-0.7)
B0 = np.float32(-0.7)
B1 = np.float32(1.5)

_BCAST = np.kron(np.eye(GR, dtype=np.float32), np.ones((1, L), np.float32))


def _body(real_hbm, fake_hbm, fw_hbm, g_hbm, out_hbm,
          rb0, rb1, fb0, fb1, gball, fwb, accb,
          sr0, sr1, sf0, sf1):
    info = plsc.get_sparse_core_info()
    nc = info.num_cores
    wid = lax.axis_index("s") * nc + lax.axis_index("c")
    bat = lax.div(wid, K_SPLIT)
    row0 = lax.rem(wid, K_SPLIT) * TROWS

    pltpu.sync_copy(fw_hbm.at[pl.ds(bat * PP, PP)], fwb)
    pltpu.sync_copy(g_hbm.at[pl.ds(bat * (PP // GR), PP // GR)], gball)
    g = []
    for c in range(CPR):
        w = fwb[pl.ds(c * L, L)]
        g.append(jnp.where(w > FZERO, ONE, FZERO))

    rbufs = (rb0, rb1)
    fbufs = (fb0, fb1)
    rsems = (sr0, sr1)
    fsems = (sf0, sf1)

    def start(ci):
        k = ci % 2
        r0 = row0 + ci * R
        cr = pltpu.async_copy(real_hbm.at[bat, pl.ds(r0, R)], rbufs[k], rsems[k])
        cf = pltpu.async_copy(fake_hbm.at[bat, pl.ds(r0, R)], fbufs[k], fsems[k])
        return cr, cf

    copies = [None] * NCHUNK
    copies[0] = start(0)

    zero = jnp.zeros((L,), jnp.float32)
    accs = (zero, zero, zero, zero)
    iota = lax.broadcasted_iota(jnp.int32, (L,), 0)

    for ci in range(NCHUNK):
        if ci + 1 < NCHUNK:
            copies[ci + 1] = start(ci + 1)
        for cp in copies[ci]:
            cp.wait()
        rb = rbufs[ci % 2]
        fb = fbufs[ci % 2]

        def rbody(r, a):
            aa = [a[0], a[1], a[2], a[3]]
            for c in range(CPR):
                aa[c % 4] = aa[c % 4] + jnp.abs(rb[r, pl.ds(c * L, L)] - C_OUT)
            return (aa[0], aa[1], aa[2], aa[3])

        accs = lax.fori_loop(0, R, rbody, accs)

        def fbody(r, a):
            a0, a1, a2, a3 = a
            i_glob = row0 + ci * R + r

            lane = lax.rem(i_glob, L)
            dv = rb[r, pl.ds(pl.multiple_of(i_glob - lane, L), L)]
            corr = jnp.abs(dv - ONE) - jnp.abs(dv - C_OUT)
            a0 = a0 + jnp.where(iota == lane, corr, FZERO)

            gi = gball[lax.div(i_glob, GR),
                       pl.ds(pl.multiple_of(lax.rem(i_glob, GR) * L, L), L)]
            av = A0 + A1 * gi
            bv = B0 + B1 * gi
            aa = [a0, a1, a2, a3]
            for c in range(CPR):
                x = fb[r, pl.ds(c * L, L)]
                t = av + bv * g[c]
                aa[c % 4] = aa[c % 4] + jnp.abs(x - t)
            return (aa[0], aa[1], aa[2], aa[3])

        accs = lax.fori_loop(0, R, fbody, accs)

    acc = (accs[0] + accs[1]) + (accs[2] + accs[3])
    accb[...] = acc
    pltpu.sync_copy(accb, out_hbm.at[wid])


def _sc_call(real, fake, fw_flat, g16):
    mesh = plsc.VectorSubcoreMesh(core_axis_name="c", subcore_axis_name="s")
    kfn = functools.partial(
        pl.kernel,
        mesh=mesh,
        out_type=jax.ShapeDtypeStruct((32, L), jnp.float32),
        scratch_types=[
            pltpu.VMEM((R, PP), jnp.float32),
            pltpu.VMEM((R, PP), jnp.float32),
            pltpu.VMEM((R, PP), jnp.float32),
            pltpu.VMEM((R, PP), jnp.float32),
            pltpu.VMEM((PP // GR, GR * L), jnp.float32),
            pltpu.VMEM((PP,), jnp.float32),
            pltpu.VMEM((L,), jnp.float32),
            pltpu.SemaphoreType.DMA,
            pltpu.SemaphoreType.DMA,
            pltpu.SemaphoreType.DMA,
            pltpu.SemaphoreType.DMA,
        ],
    )(_body)
    return kfn(real, fake, fw_flat, g16)


RB = 576
NRB = PP // RB
BB = 4


def _tc_body(fw_ref, real_ref, fake_ref, out_ref):
    b = pl.program_id(0)

    ii = lax.broadcasted_iota(jnp.int32, (RB, PP), 0)
    jj = lax.broadcasted_iota(jnp.int32, (RB, PP), 1)
    tr = jnp.where(ii == jj, ONE, C_OUT)

    s = FZERO
    for u in range(BB):
        x = real_ref[u]
        s = s + jnp.sum(jnp.abs(x - tr))
        y = fake_ref[u]
        g1 = jnp.where(fw_ref[u, 0, 0] > FZERO, ONE, FZERO)
        gj = g1.reshape(1, PP)
        gi = g1.reshape(RB, 1)
        t = A0 + A1 * (gi + gj) + np.float32(1.5) * (gi * gj)
        s = s + jnp.sum(jnp.abs(y - t))

    @pl.when(b == 0)
    def _init():
        out_ref[0, 0] = FZERO

    out_ref[0, 0] += s * np.float32(1.0 / DENOM)


def _tc_call(real, fake, fw3, gblk):
    n = B - N_SC
    return pl.pallas_call(
        _tc_body,
        grid=(n, NRB),
        in_specs=[
            pl.BlockSpec((1, 1, PP), lambda i, j: (i + N_SC, 0, 0)),
            pl.BlockSpec((1, 1, 1, RB), lambda i, j: (i + N_SC, j, 0, 0)),
            pl.BlockSpec((1, RB, PP), lambda i, j: (i + N_SC, j, 0)),
            pl.BlockSpec((1, RB, PP), lambda i, j: (i + N_SC, j, 0)),
        ],
        out_specs=pl.BlockSpec(
            (1, 1), lambda i, j: (0, 0), memory_space=pltpu.SMEM),
        out_shape=jax.ShapeDtypeStruct((1, 1), jnp.float32),
    )(fw3, gblk, real, fake)


@jax.jit
def _run(real, fake, fw):
    gvals = (fw > 0.0).astype(jnp.float32)
    fw3 = fw[:, None, :]
    gblk = gvals.reshape(B, NRB, 1, RB)
    fw_flat = fw.reshape(-1)
    g16 = gvals.reshape(B * PP // GR, GR) @ jnp.asarray(_BCAST)
    sc_parts = _sc_call(real, fake, fw_flat, g16)
    tc_sum = _tc_call(real, fake, fw3, gblk)
    return (jnp.sum(sc_parts) + tc_sum.reshape(())) / np.float32(DENOM)


@jax.jit
def _run_tc_only(real, fake, fw):
    fwr = fw.reshape(B, NRB, 1, RB)
    tc_sum = _tc_call_all(real, fake, fwr)
    return tc_sum.reshape(())


def _tc_call_all(real, fake, fw):
    return pl.pallas_call(
        _tc_body,
        grid=(B // BB,),
        in_specs=[
            pl.BlockSpec((BB, 1, 1, RB), lambda i: (i, 0, 0, 0)),
            pl.BlockSpec((BB, RB, PP), lambda i: (i, 0, 0)),
            pl.BlockSpec((BB, RB, PP), lambda i: (i, 0, 0)),
        ],
        out_specs=pl.BlockSpec(
            (1, 1), lambda i: (0, 0), memory_space=pltpu.SMEM),
        out_shape=jax.ShapeDtypeStruct((1, 1), jnp.float32),
    )(fw, real, fake)


def kernel(correlation_map_real, correlation_map_fake, fake_weight):
    return _run_tc_only(correlation_map_real, correlation_map_fake, fake_weight)
